# baseline (speedup 1.0000x reference)
import functools

import jax
import jax.numpy as jnp
from jax import lax
from jax.experimental import pallas as pl
from jax.experimental.pallas import tpu as pltpu
from jax.experimental.pallas import tpu_sc as plsc

BATCH = 16384
N_FIELDS = 26
NC = 2
NS = 16
L = 16
NW = NC * NS
B_PER_W = BATCH // NW
IDX_PER_W = B_PER_W * N_FIELDS
NCHUNK = 4
CH_COLS = B_PER_W // NCHUNK
CH_IDX = CH_COLS * N_FIELDS
CH_GROUPS = CH_COLS // L

_mesh = plsc.VectorSubcoreMesh(
    core_axis_name="c", subcore_axis_name="s", num_cores=NC, num_subcores=NS
)


@functools.partial(
    pl.kernel,
    out_type=jax.ShapeDtypeStruct((BATCH,), jnp.float32),
    mesh=_mesh,
    scratch_types=[
        pltpu.VMEM((IDX_PER_W,), jnp.int32),
        pltpu.VMEM((IDX_PER_W,), jnp.float32),
        pltpu.VMEM((B_PER_W,), jnp.float32),
        pltpu.VMEM((L,), jnp.float32),
        [pltpu.SemaphoreType.DMA for _ in range(NCHUNK)],
        [pltpu.SemaphoreType.DMA for _ in range(2)],
    ],
    compiler_params=pltpu.CompilerParams(
        needs_layout_passes=False, use_tc_tiling_on_sc=True
    ),
)
def _lr_kernel(
    idx_hbm, w_hbm, b_hbm, out_hbm, idx_v, vals_v, out_v, b_v, isems, gsems
):
    wid = lax.axis_index("s") * NC + lax.axis_index("c")
    base = wid * B_PER_W
    pltpu.sync_copy(b_hbm, b_v)

    def fire_idx(s):
        return [
            pltpu.async_copy(
                idx_hbm.at[f, pl.ds(base + s * CH_COLS, CH_COLS)],
                idx_v.at[pl.ds(s * CH_IDX + f * CH_COLS, CH_COLS)],
                isems[s],
            )
            for f in range(N_FIELDS)
        ]

    def fire_gather(s):
        return pltpu.async_copy(
            w_hbm.at[0].at[idx_v.at[pl.ds(s * CH_IDX, CH_IDX)]],
            vals_v.at[pl.ds(s * CH_IDX, CH_IDX)],
            gsems[s % 2],
        )

    idx_d = {s: fire_idx(s) for s in range(min(2, NCHUNK))}
    gather_d = {}
    for c in idx_d.pop(0):
        c.wait()
    gather_d[0] = fire_gather(0)

    bvec = b_v[...]

    for s in range(NCHUNK):
        if s + 2 < NCHUNK:
            idx_d[s + 2] = fire_idx(s + 2)
        if s + 1 < NCHUNK:
            for c in idx_d.pop(s + 1):
                c.wait()
            gather_d[s + 1] = fire_gather(s + 1)
        gather_d.pop(s).wait()

        def body(g, carry, s=s):
            accs = [bvec, 0.0, 0.0]
            for f in range(N_FIELDS):
                accs[f % 3] = accs[f % 3] + vals_v[
                    pl.ds(s * CH_IDX + f * CH_COLS + g * L, L)
                ]
            acc = (accs[0] + accs[1]) + accs[2]
            y = 1.0 / (1.0 + jnp.exp(-acc))
            out_v[pl.ds(s * CH_COLS + g * L, L)] = y
            return carry

        lax.fori_loop(0, CH_GROUPS, body, 0)

    pltpu.sync_copy(out_v, out_hbm.at[pl.ds(base, B_PER_W)])


def kernel(indices, w, b):
    idx_t = indices.T.astype(jnp.int32)
    w_t = w.T.astype(jnp.float32)
    b16 = jnp.broadcast_to(b.astype(jnp.float32), (L,))
    return _lr_kernel(idx_t, w_t, b16)


# speedup vs baseline: 1.0238x; 1.0238x over previous
import functools

import jax
import jax.numpy as jnp
from jax import lax
from jax.experimental import pallas as pl
from jax.experimental.pallas import tpu as pltpu
from jax.experimental.pallas import tpu_sc as plsc

BATCH = 16384
N_FIELDS = 26
NC = 2
NS = 16
L = 16
NW = NC * NS
B_PER_W = BATCH // NW
IDX_PER_W = B_PER_W * N_FIELDS
GROUPS = B_PER_W // L

_mesh = plsc.VectorSubcoreMesh(
    core_axis_name="c", subcore_axis_name="s", num_cores=NC, num_subcores=NS
)


@functools.partial(
    pl.kernel,
    out_type=jax.ShapeDtypeStruct((BATCH,), jnp.float32),
    mesh=_mesh,
    scratch_types=[
        pltpu.VMEM((IDX_PER_W,), jnp.int32),
        pltpu.VMEM((IDX_PER_W,), jnp.float32),
        pltpu.VMEM((B_PER_W,), jnp.float32),
        pltpu.VMEM((L,), jnp.float32),
        pltpu.SemaphoreType.DMA,
    ],
    compiler_params=pltpu.CompilerParams(
        needs_layout_passes=False, use_tc_tiling_on_sc=True
    ),
)
def _lr_kernel(idx_hbm, w_hbm, b_hbm, out_hbm, idx_v, vals_v, out_v, b_v, sem):
    wid = lax.axis_index("s") * NC + lax.axis_index("c")
    base = wid * B_PER_W
    pltpu.sync_copy(b_hbm, b_v)
    idx_copies = [
        pltpu.async_copy(
            idx_hbm.at[f, pl.ds(base, B_PER_W)],
            idx_v.at[pl.ds(f * B_PER_W, B_PER_W)],
            sem,
        )
        for f in range(N_FIELDS)
    ]
    for c in idx_copies:
        c.wait()
    pltpu.async_copy(w_hbm.at[0].at[idx_v], vals_v, sem).wait()

    bvec = b_v[...]

    def body(g, carry):
        accs = [bvec, 0.0, 0.0]
        for f in range(N_FIELDS):
            accs[f % 3] = accs[f % 3] + vals_v[pl.ds(f * B_PER_W + g * L, L)]
        acc = (accs[0] + accs[1]) + accs[2]
        y = 1.0 / (1.0 + jnp.exp(-acc))
        out_v[pl.ds(g * L, L)] = y
        return carry

    lax.fori_loop(0, GROUPS, body, 0)
    pltpu.sync_copy(out_v, out_hbm.at[pl.ds(base, B_PER_W)])


def kernel(indices, w, b):
    idx_t = indices.T.astype(jnp.int32)
    w_t = w.T.astype(jnp.float32)
    b16 = jnp.broadcast_to(b.astype(jnp.float32), (L,))
    return _lr_kernel(idx_t, w_t, b16)
